# baseline (speedup 1.0000x reference)
import jax
import jax.numpy as jnp
from jax.experimental import pallas as pl
from jax.experimental.pallas import tpu as pltpu

_BM = 200


def _contract_t(t, w):
    return jax.lax.dot_general(
        t, w, (((1,), (1,)), ((), ())), preferred_element_type=jnp.float32
    )


def _fused_kernel(adj_ref, x_ref, w0_ref, w1_ref, wss_ref, out_ref, xs_ref, h_ref):
    p = pl.program_id(0)
    i = pl.program_id(1)

    @pl.when(p == 0)
    def _phase0():
        t = jnp.dot(adj_ref[...], x_ref[...], preferred_element_type=jnp.float32)
        h_ref[pl.ds(i * _BM, _BM), :] = jnp.maximum(_contract_t(t, w0_ref[...]), 0.0)

    @pl.when(p == 1)
    def _phase1():
        t2 = jnp.dot(adj_ref[...], h_ref[...], preferred_element_type=jnp.float32)
        out_ref[...] = _contract_t(t2, w1_ref[...])
        xs_ref[...] = _contract_t(t2, wss_ref[...])


@jax.jit
def kernel(x, adj, W0, W1, Wss):
    n, d = x.shape
    bm = _BM
    ss = Wss.shape[0]
    grid = (2, n // bm)
    adj_spec = pl.BlockSpec((bm, n), lambda p, i: (i, 0))
    full_spec = pl.BlockSpec((n, d), lambda p, i: (0, 0))
    w_spec = pl.BlockSpec((d, d), lambda p, i: (0, 0))
    out_spec = pl.BlockSpec((bm, d), lambda p, i: (p * i, 0))
    xs_spec = pl.BlockSpec((bm, ss), lambda p, i: (p * i, 0))

    out, xs = pl.pallas_call(
        _fused_kernel,
        grid=grid,
        in_specs=[adj_spec, full_spec, w_spec, w_spec,
                  pl.BlockSpec((ss, d), lambda p, i: (0, 0))],
        out_specs=[out_spec, xs_spec],
        out_shape=[
            jax.ShapeDtypeStruct((n, d), jnp.float32),
            jax.ShapeDtypeStruct((n, ss), jnp.float32),
        ],
        scratch_shapes=[pltpu.VMEM((n, d), jnp.float32)],
        compiler_params=pltpu.CompilerParams(
            dimension_semantics=("arbitrary", "arbitrary")
        ),
    )(adj, x, W0, W1, Wss)

    return (out, xs)


# speedup vs baseline: 1.0416x; 1.0416x over previous
import jax
import jax.numpy as jnp
from jax.experimental import pallas as pl
from jax.experimental.pallas import tpu as pltpu

_BM = 400


def _contract_t(t, w):
    return jax.lax.dot_general(
        t, w, (((1,), (1,)), ((), ())), preferred_element_type=jnp.float32
    )


def _fused_kernel(adj_ref, x_ref, w0_ref, w1_ref, wss_ref, out_ref, xs_ref, h_ref):
    p = pl.program_id(0)
    i = pl.program_id(1)

    @pl.when(p == 0)
    def _phase0():
        t = jnp.dot(adj_ref[...], x_ref[...], preferred_element_type=jnp.float32)
        h_ref[pl.ds(i * _BM, _BM), :] = jnp.maximum(_contract_t(t, w0_ref[...]), 0.0)

    @pl.when(p == 1)
    def _phase1():
        t2 = jnp.dot(adj_ref[...], h_ref[...], preferred_element_type=jnp.float32)
        out_ref[...] = _contract_t(t2, w1_ref[...])
        xs_ref[...] = _contract_t(t2, wss_ref[...])


@jax.jit
def kernel(x, adj, W0, W1, Wss):
    n, d = x.shape
    bm = _BM
    ss = Wss.shape[0]
    grid = (2, n // bm)
    adj_spec = pl.BlockSpec((bm, n), lambda p, i: (i, 0))
    full_spec = pl.BlockSpec((n, d), lambda p, i: (0, 0))
    w_spec = pl.BlockSpec((d, d), lambda p, i: (0, 0))
    out_spec = pl.BlockSpec((bm, d), lambda p, i: (p * i, 0))
    xs_spec = pl.BlockSpec((bm, ss), lambda p, i: (p * i, 0))

    out, xs = pl.pallas_call(
        _fused_kernel,
        grid=grid,
        in_specs=[adj_spec, full_spec, w_spec, w_spec,
                  pl.BlockSpec((ss, d), lambda p, i: (0, 0))],
        out_specs=[out_spec, xs_spec],
        out_shape=[
            jax.ShapeDtypeStruct((n, d), jnp.float32),
            jax.ShapeDtypeStruct((n, ss), jnp.float32),
        ],
        scratch_shapes=[pltpu.VMEM((n, d), jnp.float32)],
        compiler_params=pltpu.CompilerParams(
            dimension_semantics=("arbitrary", "arbitrary")
        ),
    )(adj, x, W0, W1, Wss)

    return (out, xs)


# speedup vs baseline: 1.2197x; 1.1709x over previous
import jax
import jax.numpy as jnp
from jax.experimental import pallas as pl
from jax.experimental.pallas import tpu as pltpu

_BM = 400


def _contract_t(t, w):
    return jax.lax.dot_general(
        t, w, (((1,), (1,)), ((), ())), preferred_element_type=jnp.float32
    )


def _pass1_kernel(adj_ref, x_ref, w0_ref, h_ref, adj8_ref):
    adj_b = adj_ref[...].astype(jnp.bfloat16)
    adj8_ref[...] = adj_b.astype(jnp.float8_e4m3fn)
    t = jnp.dot(adj_b, x_ref[...], preferred_element_type=jnp.float32)
    h = jnp.maximum(_contract_t(t, w0_ref[...]), 0.0)
    h_ref[...] = h.astype(jnp.float8_e4m3fn)


def _pass2_kernel(adj8_ref, h_ref, w1_ref, wss_ref, out_ref, xs_ref):
    t2 = jnp.dot(adj8_ref[...], h_ref[...], preferred_element_type=jnp.float32)
    out_ref[...] = _contract_t(t2, w1_ref[...])
    xs_ref[...] = _contract_t(t2, wss_ref[...])


@jax.jit
def kernel(x, adj, W0, W1, Wss):
    n, d = x.shape
    bm = _BM
    ss = Wss.shape[0]
    grid = (n // bm,)
    adj_spec = pl.BlockSpec((bm, n), lambda i: (i, 0))
    full_spec = pl.BlockSpec((n, d), lambda i: (0, 0))
    w_spec = pl.BlockSpec((d, d), lambda i: (0, 0))
    row_spec = pl.BlockSpec((bm, d), lambda i: (i, 0))
    params = pltpu.CompilerParams(dimension_semantics=("arbitrary",))

    h8, adj8 = pl.pallas_call(
        _pass1_kernel,
        grid=grid,
        in_specs=[adj_spec, full_spec, w_spec],
        out_specs=[row_spec, adj_spec],
        out_shape=[
            jax.ShapeDtypeStruct((n, d), jnp.float8_e4m3fn),
            jax.ShapeDtypeStruct((n, n), jnp.float8_e4m3fn),
        ],
        compiler_params=params,
    )(adj, x.astype(jnp.bfloat16), W0)

    out, xs = pl.pallas_call(
        _pass2_kernel,
        grid=grid,
        in_specs=[adj_spec, full_spec, w_spec,
                  pl.BlockSpec((ss, d), lambda i: (0, 0))],
        out_specs=[row_spec, pl.BlockSpec((bm, ss), lambda i: (i, 0))],
        out_shape=[
            jax.ShapeDtypeStruct((n, d), jnp.float32),
            jax.ShapeDtypeStruct((n, ss), jnp.float32),
        ],
        compiler_params=params,
    )(adj8, h8, W1, Wss)

    return (out, xs)


# speedup vs baseline: 1.2828x; 1.0517x over previous
import jax
import jax.numpy as jnp
from jax.experimental import pallas as pl
from jax.experimental.pallas import tpu as pltpu

_BM1 = 200
_BM2 = 1000
_NBUF = 3


def _contract_t(t, w):
    return jax.lax.dot_general(
        t, w, (((1,), (1,)), ((), ())), preferred_element_type=jnp.float32
    )


def _pass1_kernel(adj_hbm, x_ref, w0_ref, h_ref, adj8_ref, buf, sem):
    i = pl.program_id(0)
    nsteps = pl.num_programs(0)

    def _fetch(blk, slot):
        return pltpu.make_async_copy(
            adj_hbm.at[pl.ds(blk * _BM1, _BM1), :], buf.at[slot], sem.at[slot]
        )

    @pl.when(i == 0)
    def _prologue():
        for s in range(_NBUF):
            _fetch(s, s).start()

    @pl.when((i > 0) & (i + _NBUF - 1 < nsteps))
    def _next():
        blk = i + _NBUF - 1
        _fetch(blk, blk % _NBUF).start()

    slot = i % _NBUF
    _fetch(i, slot).wait()
    adj_b = buf[slot].astype(jnp.bfloat16)
    adj8_ref[...] = adj_b.astype(jnp.float8_e4m3fn)
    t = jnp.dot(adj_b, x_ref[...], preferred_element_type=jnp.float32)
    h = jnp.maximum(_contract_t(t, w0_ref[...]), 0.0)
    h_ref[...] = h.astype(jnp.float8_e4m3fn)


def _pass2_kernel(adj8_ref, h_ref, w1_ref, wss_ref, out_ref, xs_ref):
    t2 = jnp.dot(adj8_ref[...], h_ref[...], preferred_element_type=jnp.float32)
    out_ref[...] = _contract_t(t2, w1_ref[...])
    xs_ref[...] = _contract_t(t2, wss_ref[...])


@jax.jit
def kernel(x, adj, W0, W1, Wss):
    n, d = x.shape
    ss = Wss.shape[0]
    full_spec = pl.BlockSpec((n, d), lambda i: (0, 0))
    w_spec = pl.BlockSpec((d, d), lambda i: (0, 0))
    params = pltpu.CompilerParams(dimension_semantics=("arbitrary",))

    h8, adj8 = pl.pallas_call(
        _pass1_kernel,
        grid=(n // _BM1,),
        in_specs=[pl.BlockSpec(memory_space=pl.ANY), full_spec, w_spec],
        out_specs=[
            pl.BlockSpec((_BM1, d), lambda i: (i, 0)),
            pl.BlockSpec((_BM1, n), lambda i: (i, 0)),
        ],
        out_shape=[
            jax.ShapeDtypeStruct((n, d), jnp.float8_e4m3fn),
            jax.ShapeDtypeStruct((n, n), jnp.float8_e4m3fn),
        ],
        scratch_shapes=[
            pltpu.VMEM((_NBUF, _BM1, n), jnp.float32),
            pltpu.SemaphoreType.DMA((_NBUF,)),
        ],
        compiler_params=params,
    )(adj, x.astype(jnp.bfloat16), W0)

    out, xs = pl.pallas_call(
        _pass2_kernel,
        grid=(n // _BM2,),
        in_specs=[pl.BlockSpec((_BM2, n), lambda i: (i, 0)), full_spec, w_spec,
                  pl.BlockSpec((ss, d), lambda i: (0, 0))],
        out_specs=[pl.BlockSpec((_BM2, d), lambda i: (i, 0)),
                   pl.BlockSpec((_BM2, ss), lambda i: (i, 0))],
        out_shape=[
            jax.ShapeDtypeStruct((n, d), jnp.float32),
            jax.ShapeDtypeStruct((n, ss), jnp.float32),
        ],
        compiler_params=params,
    )(adj8, h8, W1, Wss)

    return (out, xs)
